# baseline (speedup 1.0000x reference)
import functools

import jax
import jax.numpy as jnp
from jax.experimental import pallas as pl
from jax.experimental.pallas import tpu as pltpu

T = 4
_SELU_SCALE = 1.0507009873554804934193349852946
_SELU_ALPHA = 1.6732632423543772848170429916717


def _selu(x):
    neg = jnp.minimum(x, 0.0)
    pos = jnp.maximum(x, 0.0)
    return _SELU_SCALE * (pos + _SELU_ALPHA * (jnp.exp(neg) - 1.0))


def _readout_body(state_ref, wr1_ref, br1_ref, wr2_ref, br2_ref, wout_ref, bout_ref, out_ref, acc_ref):
    i = pl.program_id(0)
    n = pl.num_programs(0)

    @pl.when(i == 0)
    def _():
        acc_ref[...] = jnp.zeros_like(acc_ref)

    acc_ref[...] += jnp.sum(state_ref[...], axis=0, keepdims=True)

    @pl.when(i == n - 1)
    def _():
        feature = acc_ref[...]
        h = _selu(feature @ wr1_ref[...].T + br1_ref[...])
        h = _selu(h @ wr2_ref[...].T + br2_ref[...])
        out_ref[...] = jnp.sum(h * wout_ref[...], axis=1, keepdims=True) + bout_ref[...]


def _readout(state, W_r1, b_r1, W_r2, b_r2, W_out, b_out):
    N, F = state.shape
    R = W_r1.shape[0]
    BLK = 4000
    grid = N // BLK
    return pl.pallas_call(
        _readout_body,
        grid=(grid,),
        in_specs=[
            pl.BlockSpec((BLK, F), lambda i: (i, 0)),
            pl.BlockSpec((R, F), lambda i: (0, 0)),
            pl.BlockSpec((1, R), lambda i: (0, 0)),
            pl.BlockSpec((R, R), lambda i: (0, 0)),
            pl.BlockSpec((1, R), lambda i: (0, 0)),
            pl.BlockSpec((1, R), lambda i: (0, 0)),
            pl.BlockSpec((1, 1), lambda i: (0, 0)),
        ],
        out_specs=pl.BlockSpec((1, 1), lambda i: (0, 0)),
        out_shape=jax.ShapeDtypeStruct((1, 1), jnp.float32),
        scratch_shapes=[pltpu.VMEM((1, F), jnp.float32)],
    )(state, W_r1, b_r1.reshape(1, R), W_r2, b_r2.reshape(1, R),
      W_out, b_out.reshape(1, 1)).reshape(1)


def kernel(link_state, first, second, state_dim, W_msg, b_msg, W_ih, W_hh, b_ih, b_hh, W_r1, b_r1, W_r2, b_r2, W_out, b_out):
    state = link_state
    n_nodes = state.shape[0]
    for _ in range(T):
        main_edges = jnp.take(state, first, axis=0)
        neigh_edges = jnp.take(state, second, axis=0)
        edges_concat = jnp.concatenate([main_edges, neigh_edges], axis=1)
        m = jax.nn.selu(edges_concat @ W_msg.T + b_msg)
        agg = jax.ops.segment_sum(m, second, num_segments=n_nodes)
        gi = agg @ W_ih.T + b_ih
        gh = state @ W_hh.T + b_hh
        i_r, i_z, i_n = jnp.split(gi, 3, axis=1)
        h_r, h_z, h_n = jnp.split(gh, 3, axis=1)
        r = jax.nn.sigmoid(i_r + h_r)
        z = jax.nn.sigmoid(i_z + h_z)
        n = jnp.tanh(i_n + r * h_n)
        state = (1.0 - z) * n + z * state
    return _readout(state, W_r1, b_r1, W_r2, b_r2, W_out, b_out)


# speedup vs baseline: 4.6544x; 4.6544x over previous
import functools

import jax
import jax.numpy as jnp
from jax import lax
from jax.experimental import pallas as pl
from jax.experimental.pallas import tpu as pltpu
from jax.experimental.pallas import tpu_sc as plsc

T = 4
N = 100000
E = 3200000
F = 20
NPAD = 100096
CHUNK = 128
NW = 32
CH_PER_W = 784
NCHUNKS = NW * CH_PER_W
EPAD = NCHUNKS * CHUNK
CH_PER_S = NCHUNKS // 16
G = 2
NGROUPS = CH_PER_S // G
WR = 32
QR = NPAD // 4
AGR = QR + 8
TPSQ = QR // 16
BLK = 4000

_SELU_SCALE = 1.0507009873554804934193349852946
_SELU_ALPHA = 1.6732632423543772848170429916717


def _selu(x):
    neg = jnp.minimum(x, 0.0)
    pos = jnp.maximum(x, 0.0)
    return _SELU_SCALE * (pos + _SELU_ALPHA * (jnp.exp(neg) - 1.0))


def _sc_body(p_hbm, q_hbm, f_hbm, s_hbm, out_hbm,
             fidx, sidx, rP, rQ,
             agg, sem_g0, sem_g1, sem_s0, sem_s1,
             sem_i0, sem_i1, sem_i2, sem_i3):
    c = lax.axis_index("c")
    s = lax.axis_index("s")
    row0 = s * TPSQ
    start = s * CH_PER_S

    sem_g = (sem_g0, sem_g1)
    sem_s = (sem_s0, sem_s1)
    sem_i = (sem_i0, sem_i1, sem_i2, sem_i3)

    for k in range(2):
        q = 2 * c + k
        base = q * QR

        def zrow_body(r, carry):
            zv = jnp.zeros((16,), jnp.float32)
            rP[0, 0, r, pl.ds(0, 16)] = zv
            rP[0, 0, r, pl.ds(16, 16)] = zv
            return carry

        lax.fori_loop(0, CHUNK, zrow_body, 0, unroll=2)
        for i in range(12):
            pltpu.sync_copy(rP.at[0, 0],
                            agg.at[pl.ds(row0 + i * CHUNK, CHUNK), :])
        pltpu.sync_copy(rP.at[0, 0, pl.ds(0, 28)],
                        agg.at[pl.ds(row0 + 12 * CHUNK, 28), :])

        @pl.when(s == 0)
        def _():
            pltpu.sync_copy(rP.at[0, 0, pl.ds(0, 8)],
                            agg.at[pl.ds(QR, 8), :])

        plsc.subcore_barrier()

        def stage_idx(g, sx):
            for j in range(G):
                c0 = (start + g * G + j) * CHUNK
                pltpu.async_copy(f_hbm.at[pl.ds(c0, CHUNK)], fidx.at[sx, j],
                                 sem_i[sx])
                pltpu.async_copy(s_hbm.at[pl.ds(c0, CHUNK)], sidx.at[sx, j],
                                 sem_i[sx])

        def wait_idx(g, sx):
            for j in range(G):
                c0 = (start + g * G + j) * CHUNK
                pltpu.make_async_copy(f_hbm.at[pl.ds(c0, CHUNK)],
                                      fidx.at[sx, j], sem_i[sx]).wait()
                pltpu.make_async_copy(s_hbm.at[pl.ds(c0, CHUNK)],
                                      sidx.at[sx, j], sem_i[sx]).wait()

        def gathers(sx, b):
            for j in range(G):
                pltpu.async_copy(p_hbm.at[fidx.at[sx, j]], rP.at[b, j],
                                 sem_g[b])
                pltpu.async_copy(q_hbm.at[sidx.at[sx, j]], rQ.at[b, j],
                                 sem_g[b])

        def drain_scatters(sx, b):
            for j in range(G):
                pltpu.make_async_copy(rP.at[b, j], agg.at[sidx.at[sx, j]],
                                      sem_s[b]).wait()

        def do_group(g, b, sx):
            nb = 1 - b
            nsx = (sx + 1) % 4

            @pl.when(g + 1 < NGROUPS)
            def _():
                @pl.when(g >= 1)
                def _():
                    drain_scatters((sx + 3) % 4, nb)
                wait_idx(g + 1, nsx)
                gathers(nsx, nb)

            @pl.when(g + 2 < NGROUPS)
            def _():
                stage_idx(g + 2, (sx + 2) % 4)

            for j in range(G):
                pltpu.make_async_copy(p_hbm.at[fidx.at[sx, j]], rP.at[b, j],
                                      sem_g[b]).wait()
                pltpu.make_async_copy(q_hbm.at[sidx.at[sx, j]], rQ.at[b, j],
                                      sem_g[b]).wait()

            for j in range(G):
                def idx_body(w, carry):
                    o = w * 16
                    v = sidx[sx, j, pl.ds(o, 16)]
                    inr = (v >= base) & (v < base + QR)
                    sidx[sx, j, pl.ds(o, 16)] = jnp.where(inr, v - base, QR)
                    return carry

                lax.fori_loop(0, CHUNK // 16, idx_body, 0, unroll=2)

                def row_body(r, carry):
                    p0 = rP[b, j, r, pl.ds(0, 16)]
                    p1 = rP[b, j, r, pl.ds(16, 16)]
                    q0 = rQ[b, j, r, pl.ds(0, 16)]
                    q1 = rQ[b, j, r, pl.ds(16, 16)]
                    rP[b, j, r, pl.ds(0, 16)] = _selu(p0 + q0)
                    rP[b, j, r, pl.ds(16, 16)] = _selu(p1 + q1)
                    return carry

                lax.fori_loop(0, CHUNK, row_body, 0, unroll=2)
                pltpu.async_copy(rP.at[b, j], agg.at[sidx.at[sx, j]],
                                 sem_s[b], add=True)

        stage_idx(0, 0)
        stage_idx(1, 1)
        wait_idx(0, 0)
        gathers(0, 0)

        def it_body(it, carry):
            g0 = 4 * it
            do_group(g0, 0, 0)
            do_group(g0 + 1, 1, 1)
            do_group(g0 + 2, 0, 2)
            do_group(g0 + 3, 1, 3)
            return carry

        lax.fori_loop(0, NGROUPS // 4, it_body, 0)
        drain_scatters(2, 0)
        drain_scatters(3, 1)

        plsc.subcore_barrier()
        pltpu.sync_copy(agg.at[pl.ds(row0, TPSQ), :],
                        out_hbm.at[pl.ds(base + row0, TPSQ), :])


def _sc_pass(Ppad, Qpad, firstp, secondp):
    mesh = plsc.VectorSubcoreMesh(core_axis_name="c", subcore_axis_name="s")
    f = pl.kernel(
        _sc_body,
        out_type=jax.ShapeDtypeStruct((NPAD, WR), jnp.float32),
        mesh=mesh,
        scratch_types=[
            pltpu.VMEM((4, G, CHUNK), jnp.int32),
            pltpu.VMEM((4, G, CHUNK), jnp.int32),
            pltpu.VMEM((2, G, CHUNK, WR), jnp.float32),
            pltpu.VMEM((2, G, CHUNK, WR), jnp.float32),
            pltpu.VMEM_SHARED((AGR, WR), jnp.float32),
            pltpu.SemaphoreType.DMA,
            pltpu.SemaphoreType.DMA,
            pltpu.SemaphoreType.DMA,
            pltpu.SemaphoreType.DMA,
            pltpu.SemaphoreType.DMA,
            pltpu.SemaphoreType.DMA,
            pltpu.SemaphoreType.DMA,
            pltpu.SemaphoreType.DMA,
        ],
        compiler_params=pltpu.CompilerParams(use_tc_tiling_on_sc=False),
    )
    return f(Ppad, Qpad, firstp, secondp)


def _prep_body(state_ref, waT_ref, wbT_ref, bmsg_ref, p_ref, q_ref):
    st = state_ref[...]
    p_ref[...] = jnp.dot(st, waT_ref[...],
                         preferred_element_type=jnp.float32) + bmsg_ref[...]
    q_ref[...] = jnp.dot(st, wbT_ref[...], preferred_element_type=jnp.float32)


def _prep(state, waT, wbT, bmsg):
    grid = N // BLK
    return pl.pallas_call(
        _prep_body,
        grid=(grid,),
        in_specs=[
            pl.BlockSpec((BLK, F), lambda i: (i, 0)),
            pl.BlockSpec((F, F), lambda i: (0, 0)),
            pl.BlockSpec((F, F), lambda i: (0, 0)),
            pl.BlockSpec((1, F), lambda i: (0, 0)),
        ],
        out_specs=[
            pl.BlockSpec((BLK, F), lambda i: (i, 0)),
            pl.BlockSpec((BLK, F), lambda i: (i, 0)),
        ],
        out_shape=[
            jax.ShapeDtypeStruct((N, F), jnp.float32),
            jax.ShapeDtypeStruct((N, F), jnp.float32),
        ],
    )(state, waT, wbT, bmsg)


def _gru_body(state_ref, p_ref, wihT_ref, whhT_ref, bih_ref,
              bhh_ref, waT_ref, wbT_ref, bmsg_ref, ns_ref, pn_ref, qn_ref):
    st = state_ref[...]
    agg = p_ref[:, :F]
    gi = jnp.dot(agg, wihT_ref[...],
                 preferred_element_type=jnp.float32) + bih_ref[...]
    gh = jnp.dot(st, whhT_ref[...],
                 preferred_element_type=jnp.float32) + bhh_ref[...]
    r = jax.nn.sigmoid(gi[:, :F] + gh[:, :F])
    z = jax.nn.sigmoid(gi[:, F:2 * F] + gh[:, F:2 * F])
    n = jnp.tanh(gi[:, 2 * F:] + r * gh[:, 2 * F:])
    ns = (1.0 - z) * n + z * st
    ns_ref[...] = ns
    pn_ref[...] = jnp.dot(ns, waT_ref[...],
                          preferred_element_type=jnp.float32) + bmsg_ref[...]
    qn_ref[...] = jnp.dot(ns, wbT_ref[...], preferred_element_type=jnp.float32)


def _gru(state, partial, wihT, whhT, bih, bhh, waT, wbT, bmsg):
    grid = N // BLK
    return pl.pallas_call(
        _gru_body,
        grid=(grid,),
        in_specs=[
            pl.BlockSpec((BLK, F), lambda i: (i, 0)),
            pl.BlockSpec((BLK, WR), lambda i: (i, 0)),
            pl.BlockSpec((F, 3 * F), lambda i: (0, 0)),
            pl.BlockSpec((F, 3 * F), lambda i: (0, 0)),
            pl.BlockSpec((1, 3 * F), lambda i: (0, 0)),
            pl.BlockSpec((1, 3 * F), lambda i: (0, 0)),
            pl.BlockSpec((F, F), lambda i: (0, 0)),
            pl.BlockSpec((F, F), lambda i: (0, 0)),
            pl.BlockSpec((1, F), lambda i: (0, 0)),
        ],
        out_specs=[
            pl.BlockSpec((BLK, F), lambda i: (i, 0)),
            pl.BlockSpec((BLK, F), lambda i: (i, 0)),
            pl.BlockSpec((BLK, F), lambda i: (i, 0)),
        ],
        out_shape=[
            jax.ShapeDtypeStruct((N, F), jnp.float32),
            jax.ShapeDtypeStruct((N, F), jnp.float32),
            jax.ShapeDtypeStruct((N, F), jnp.float32),
        ],
    )(state, partial, wihT, whhT, bih, bhh, waT, wbT, bmsg)


def _readout_body(state_ref, wr1_ref, br1_ref, wr2_ref, br2_ref, wout_ref,
                  bout_ref, out_ref, acc_ref):
    i = pl.program_id(0)
    n = pl.num_programs(0)

    @pl.when(i == 0)
    def _():
        acc_ref[...] = jnp.zeros_like(acc_ref)

    acc_ref[...] += jnp.sum(state_ref[...], axis=0, keepdims=True)

    @pl.when(i == n - 1)
    def _():
        feature = acc_ref[...]
        h = _selu(feature @ wr1_ref[...].T + br1_ref[...])
        h = _selu(h @ wr2_ref[...].T + br2_ref[...])
        out_ref[...] = jnp.sum(h * wout_ref[...], axis=1,
                               keepdims=True) + bout_ref[...]


def _readout(state, W_r1, b_r1, W_r2, b_r2, W_out, b_out):
    R = W_r1.shape[0]
    grid = N // BLK
    return pl.pallas_call(
        _readout_body,
        grid=(grid,),
        in_specs=[
            pl.BlockSpec((BLK, F), lambda i: (i, 0)),
            pl.BlockSpec((R, F), lambda i: (0, 0)),
            pl.BlockSpec((1, R), lambda i: (0, 0)),
            pl.BlockSpec((R, R), lambda i: (0, 0)),
            pl.BlockSpec((1, R), lambda i: (0, 0)),
            pl.BlockSpec((1, R), lambda i: (0, 0)),
            pl.BlockSpec((1, 1), lambda i: (0, 0)),
        ],
        out_specs=pl.BlockSpec((1, 1), lambda i: (0, 0)),
        out_shape=jax.ShapeDtypeStruct((1, 1), jnp.float32),
        scratch_shapes=[pltpu.VMEM((1, F), jnp.float32)],
    )(state, W_r1, b_r1.reshape(1, R), W_r2, b_r2.reshape(1, R),
      W_out, b_out.reshape(1, 1)).reshape(1)


def kernel(link_state, first, second, state_dim, W_msg, b_msg, W_ih, W_hh,
           b_ih, b_hh, W_r1, b_r1, W_r2, b_r2, W_out, b_out):
    waT = W_msg[:, :F].T
    wbT = W_msg[:, F:].T
    wihT = W_ih.T
    whhT = W_hh.T
    bmsg = b_msg.reshape(1, F)
    bih = b_ih.reshape(1, 3 * F)
    bhh = b_hh.reshape(1, 3 * F)

    pad = jnp.full((EPAD - E,), N, dtype=jnp.int32)
    firstp = jnp.concatenate([first, pad])
    secondp = jnp.concatenate([second, pad])
    zcol = jnp.zeros((N, WR - F), jnp.float32)
    zrow = jnp.zeros((NPAD - N, WR), jnp.float32)

    state = link_state
    P, Q = _prep(state, waT, wbT, bmsg)

    def round_body(carry, _):
        state, P, Q = carry
        Ppad = jnp.concatenate([jnp.concatenate([P, zcol], axis=1), zrow],
                               axis=0)
        Qpad = jnp.concatenate([jnp.concatenate([Q, zcol], axis=1), zrow],
                               axis=0)
        partial = _sc_pass(Ppad, Qpad, firstp, secondp)
        state, P, Q = _gru(state, partial, wihT, whhT, bih, bhh, waT, wbT,
                           bmsg)
        return (state, P, Q), None

    (state, P, Q), _ = lax.scan(round_body, (state, P, Q), None, length=T)
    return _readout(state, W_r1, b_r1, W_r2, b_r2, W_out, b_out)
